# Initial kernel scaffold; baseline (speedup 1.0000x reference)
import jax
import jax.numpy as jnp
from jax.experimental import pallas as pl


def kernel(embeddings, compound_labels, dose_labels):
    raise NotImplementedError("write your pallas kernel here")


# speedup vs baseline: 5.0400x; 5.0400x over previous
import functools

import jax
import jax.numpy as jnp
from jax import lax
from jax.experimental import pallas as pl
from jax.experimental.pallas import tpu as pltpu
from jax.experimental.pallas import tpu_sc as plsc

B = 16384
DIM = 64
C = 1000
D = 8
MARGIN = 0.1

TBL = 8192
NC = 2
NS = 16
NW = NC * NS
PER_TILE = B // NW
ROWS = PER_TILE // 128
SLICE = TBL // NS


def _dist_kernel(emb_ref, comp_ref, dose_ref, dist_ref, key_ref):
    emb = emb_ref[...]
    col = jnp.sum(emb, axis=1)
    s = jnp.sum(col, axis=0)
    mean = s * (1.0 / B)
    norm = jnp.sqrt(jnp.sum(mean * mean))
    u = mean / jnp.maximum(norm, 1e-12)
    dist_ref[...] = 1.0 - jnp.sum(emb * u[None, None, :], axis=2)
    key_ref[...] = dose_ref[...] * 1024 + comp_ref[...]


def _segment_kernel(dist_hbm, key_hbm, sums_out, counts_out,
                    idx_v, val_v, ones_v, stage_v, sh_sums, sh_counts):
    c = lax.axis_index("c")
    s = lax.axis_index("s")
    wid = c * NS + s

    def _zero(i, _):
        stage_v[pl.ds(i * 16, 16)] = jnp.zeros((16,), jnp.float32)
        return 0
    lax.fori_loop(0, SLICE // 16, _zero, 0)
    pltpu.sync_copy(stage_v, sh_sums.at[pl.ds(s * SLICE, SLICE)])
    pltpu.sync_copy(stage_v, sh_counts.at[pl.ds(s * SLICE, SLICE)])

    def _ones(i, _):
        ones_v[pl.ds(i * 16, 16)] = jnp.ones((16,), jnp.float32)
        return 0
    lax.fori_loop(0, 128 // 16, _ones, 0)

    pltpu.sync_copy(key_hbm.at[wid], idx_v)
    pltpu.sync_copy(dist_hbm.at[wid], val_v)

    plsc.subcore_barrier()

    for j in range(ROWS):
        pltpu.sync_copy(val_v.at[j], sh_sums.at[idx_v.at[j]], add=True)
        pltpu.sync_copy(ones_v, sh_counts.at[idx_v.at[j]], add=True)

    plsc.subcore_barrier()

    pltpu.sync_copy(sh_sums.at[pl.ds(s * SLICE, SLICE)], stage_v)
    pltpu.sync_copy(stage_v, sums_out.at[c, pl.ds(s * SLICE, SLICE)])
    pltpu.sync_copy(sh_counts.at[pl.ds(s * SLICE, SLICE)], stage_v)
    pltpu.sync_copy(stage_v, counts_out.at[c, pl.ds(s * SLICE, SLICE)])


def _loss_kernel(sums_ref, counts_ref, out_ref):
    sums = sums_ref[0] + sums_ref[1]
    counts = counts_ref[0] + counts_ref[1]
    present = counts > 0.0
    means = jnp.where(present, sums / jnp.maximum(counts, 1.0), 0.0)
    ploss = jnp.zeros((1, 1024), jnp.float32)
    pcnt = jnp.zeros((1, 1024), jnp.float32)
    for dl in range(D - 1):
        for dh in range(dl + 1, D):
            valid = present[dl:dl + 1] & present[dh:dh + 1]
            for m in range(dl + 1, dh):
                valid = valid & jnp.logical_not(present[m:m + 1])
            viol = MARGIN - (means[dh:dh + 1] - means[dl:dl + 1])
            ploss = ploss + jnp.where(valid, jnp.maximum(viol, 0.0), 0.0)
            pcnt = pcnt + valid.astype(jnp.float32)
    loss = jnp.sum(ploss)
    cnt = jnp.sum(pcnt)
    out_ref[...] = jnp.where(cnt > 0.0, loss / jnp.maximum(cnt, 1.0),
                             0.0)[None, None]


def kernel(embeddings, compound_labels, dose_labels):
    emb3 = embeddings.reshape(128, 128, DIM)
    comp = compound_labels.astype(jnp.int32).reshape(128, 128)
    dose = dose_labels.astype(jnp.int32).reshape(128, 128)

    dist, keys = pl.pallas_call(
        _dist_kernel,
        out_shape=(
            jax.ShapeDtypeStruct((128, 128), jnp.float32),
            jax.ShapeDtypeStruct((128, 128), jnp.int32),
        ),
    )(emb3, comp, dose)

    seg = functools.partial(
        pl.kernel,
        mesh=plsc.VectorSubcoreMesh(core_axis_name="c", subcore_axis_name="s"),
        out_type=(
            jax.ShapeDtypeStruct((NC, TBL), jnp.float32),
            jax.ShapeDtypeStruct((NC, TBL), jnp.float32),
        ),
        scratch_types=[
            pltpu.VMEM((ROWS, 128), jnp.int32),
            pltpu.VMEM((ROWS, 128), jnp.float32),
            pltpu.VMEM((128,), jnp.float32),
            pltpu.VMEM((SLICE,), jnp.float32),
            pltpu.VMEM_SHARED((TBL,), jnp.float32),
            pltpu.VMEM_SHARED((TBL,), jnp.float32),
        ],
    )(_segment_kernel)
    sums2, counts2 = seg(dist.reshape(NW, ROWS, 128),
                         keys.reshape(NW, ROWS, 128))

    out = pl.pallas_call(
        _loss_kernel,
        out_shape=jax.ShapeDtypeStruct((1, 1), jnp.float32),
    )(sums2.reshape(NC, D, 1024), counts2.reshape(NC, D, 1024))
    return out.reshape(())
